# baseline (speedup 1.0000x reference)
import jax
import jax.numpy as jnp
from jax import lax
from jax.experimental import pallas as pl
from jax.experimental.pallas import tpu as pltpu
from jax.experimental.pallas import tpu_sc as plsc

B = 4096
H = 15
W = 15
P = H * W
D = 16
PD = 2380
V = (PD + 1) ** 2

NC = 2
NS = 16
NW = NC * NS
IMG_PER_W = B // NW
CHUNK = 8
NCHUNK = IMG_PER_W // CHUNK
NPOS = CHUNK * P
NGRP = 113
NPAD = 1808
NIDXROW = 15
NROWS = NIDXROW * 128


def _sc_kernel(sf0_hbm, sf1_hbm, bd0_hbm, bd1_hbm, tg_hbm, out_hbm,
               sf0, sf1, bd0, bd1, vbuf, rowb, colb,
               idxb, laneb, rows, outv, sem0, sem1):
    wid = lax.axis_index("s") * NC + lax.axis_index("c")
    base_img = wid * IMG_PER_W

    iota = lax.broadcasted_iota(jnp.int32, (16,), 0)
    zeros16 = jnp.zeros((16,), jnp.int32)

    sf0[pl.ds(1792, 16)] = zeros16
    sf1[pl.ds(1792, 16)] = zeros16
    bd0[pl.ds(1792, 16)] = zeros16
    bd1[pl.ds(1792, 16)] = zeros16
    for par in range(2):
        for gg in range(1, 8):
            idxb[par, NIDXROW - 1, pl.ds(16 * gg, 16)] = zeros16

    def build_idx(d, par):
        dV = d * V

        for g in range(NGRP):
            f = 16 * g
            v = vbuf[pl.ds(f, 16)]
            fl = v + dV
            idxb[par, f // 128, pl.ds(f % 128, 16)] = \
                lax.shift_right_logical(fl, 4)
            laneb[par, pl.ds(f, 16)] = lax.bitwise_and(fl, 15)

    def fire(par, sem):
        @pl.loop(0, NIDXROW)
        def _fire(i):
            pltpu.make_async_copy(
                tg_hbm.at[idxb.at[par, i]],
                rows.at[par, pl.ds(128 * i, 128)],
                sem,
            ).start()

    tail_mask = iota < (NPOS - 16 * (NGRP - 1))

    def drain_extract(d, par, sem):
        for i in range(NIDXROW):
            pltpu.make_async_copy(
                tg_hbm.at[idxb.at[par, i]],
                rows.at[par, pl.ds(128 * i, 128)],
                sem,
            ).wait()
            ng = 8 if i < NIDXROW - 1 else 1
            for k in range(ng):
                g = 8 * i + k
                f = 16 * g
                jvec = f + iota
                lanes = laneb[par, pl.ds(f, 16)]
                val = plsc.load_gather(rows.at[par], [jvec, lanes])
                rvec = rowb[pl.ds(f, 16)] + d
                cvec = colb[pl.ds(f, 16)]
                if g == NGRP - 1:
                    plsc.store_scatter(outv, [rvec, cvec], val,
                                       mask=tail_mask)
                else:
                    plsc.store_scatter(outv, [rvec, cvec], val)

    @pl.loop(0, NCHUNK)
    def _chunk(c):
        b0 = base_img + c * CHUNK
        e0 = b0 * P

        pltpu.sync_copy(sf0_hbm.at[pl.ds(e0, NPOS)], sf0.at[pl.ds(0, NPOS)])
        pltpu.sync_copy(sf1_hbm.at[pl.ds(e0, NPOS)], sf1.at[pl.ds(0, NPOS)])
        pltpu.sync_copy(bd0_hbm.at[pl.ds(e0, NPOS)], bd0.at[pl.ds(0, NPOS)])
        pltpu.sync_copy(bd1_hbm.at[pl.ds(e0, NPOS)], bd1.at[pl.ds(0, NPOS)])

        imgv = jnp.zeros((16,), jnp.int32)
        pv = iota
        for g in range(NGRP):
            f = 16 * g
            s0 = sf0[pl.ds(f, 16)]
            s1 = sf1[pl.ds(f, 16)]
            c0 = bd0[pl.ds(f, 16)]
            c1 = bd1[pl.ds(f, 16)]
            p0 = jnp.where(c0 > 0, PD, s0)
            p1 = jnp.where(c1 > 0, PD, s1)
            vbuf[pl.ds(f, 16)] = p1 * (PD + 1) + p0
            if g == NGRP - 1:
                flat = f + iota
                im = flat // P
                pp = flat - im * P
                rowb[pl.ds(f, 16)] = im * 16
                colb[pl.ds(f, 16)] = pp
            else:
                rowb[pl.ds(f, 16)] = imgv
                colb[pl.ds(f, 16)] = pv
                pv = pv + 16
                wrap = pv >= P
                imgv = jnp.where(wrap, imgv + 16, imgv)
                pv = jnp.where(wrap, pv - P, pv)

        build_idx(0, 0)
        fire(0, sem0)

        @pl.loop(0, D // 2)
        def _dpair(dp):
            d0 = 2 * dp
            build_idx(d0 + 1, 1)
            fire(1, sem1)
            drain_extract(d0, 0, sem0)

            @pl.when(d0 + 2 < D)
            def _prefetch():
                build_idx(d0 + 2, 0)
                fire(0, sem0)

            drain_extract(d0 + 1, 1, sem1)

        @pl.loop(0, CHUNK)
        def _out(img):
            pltpu.sync_copy(
                outv.at[pl.ds(16 * img, 16)],
                out_hbm.at[b0 + img],
            )


@jax.jit
def _pcode_embed(sparse_feature_input, board_input, pcode_embedding):
    sf0 = sparse_feature_input[:, 10].reshape(B * P)
    sf1 = sparse_feature_input[:, 11].reshape(B * P)
    bd0 = board_input[:, 0].reshape(B * P)
    bd1 = board_input[:, 1].reshape(B * P)
    tg = pcode_embedding.T.reshape(V, D)
    mesh = plsc.VectorSubcoreMesh(core_axis_name="c", subcore_axis_name="s")
    run = pl.kernel(
        _sc_kernel,
        out_type=jax.ShapeDtypeStruct((B, D, P), jnp.float32),
        mesh=mesh,
        scratch_types=[
            pltpu.VMEM((NPAD,), jnp.int32),
            pltpu.VMEM((NPAD,), jnp.int32),
            pltpu.VMEM((NPAD,), jnp.int32),
            pltpu.VMEM((NPAD,), jnp.int32),
            pltpu.VMEM((NPAD,), jnp.int32),
            pltpu.VMEM((NPAD,), jnp.int32),
            pltpu.VMEM((NPAD,), jnp.int32),
            pltpu.VMEM((2, NIDXROW, 128), jnp.int32),
            pltpu.VMEM((2, NPAD), jnp.int32),
            pltpu.VMEM((2, NROWS, D), jnp.float32),
            pltpu.VMEM((CHUNK * D, P), jnp.float32),
            pltpu.SemaphoreType.DMA,
            pltpu.SemaphoreType.DMA,
        ],
        compiler_params=pltpu.CompilerParams(
            use_tc_tiling_on_sc=False, needs_layout_passes=False
        ),
    )
    out = run(sf0, sf1, bd0, bd1, tg)
    return out.reshape(B, D, H, W)


def kernel(sparse_feature_dim, sparse_feature_input, board_input, pcode_embedding):
    del sparse_feature_dim
    return _pcode_embed(sparse_feature_input, board_input, pcode_embedding)


# speedup vs baseline: 3.1979x; 3.1979x over previous
import jax
import jax.numpy as jnp
from jax import lax
from jax.experimental import pallas as pl
from jax.experimental.pallas import tpu as pltpu
from jax.experimental.pallas import tpu_sc as plsc

B = 4096
H = 15
W = 15
P = H * W
D = 16
PD = 2380
VOCAB = (PD + 1) ** 2

NC = 2
NS = 16
NW = NC * NS
IMG_PER_W = B // NW
CHUNK = 8
NCHUNK = IMG_PER_W // CHUNK
NPOS = CHUNK * P
NGRP = 113
NPAD = 1808
NIDXROW = 15
NROWS = NIDXROW * 128
NT = 15


def _sc_kernel(sf0_hbm, sf1_hbm, bd0_hbm, bd1_hbm, tab_hbm, out_hbm,
               sf0, sf1, bd0, bd1, idxb, rows, outv, sem):
    wid = lax.axis_index("s") * NC + lax.axis_index("c")
    base_img = wid * IMG_PER_W

    iota = lax.broadcasted_iota(jnp.int32, (16,), 0)
    zeros16 = jnp.zeros((16,), jnp.int32)

    sf0[pl.ds(1792, 16)] = zeros16
    sf1[pl.ds(1792, 16)] = zeros16
    bd0[pl.ds(1792, 16)] = zeros16
    bd1[pl.ds(1792, 16)] = zeros16
    for gg in range(1, 8):
        idxb[NIDXROW - 1, pl.ds(16 * gg, 16)] = zeros16

    @pl.loop(0, NCHUNK)
    def _chunk(c):
        b0 = base_img + c * CHUNK
        e0 = b0 * P

        pltpu.sync_copy(sf0_hbm.at[pl.ds(e0, NPOS)], sf0.at[pl.ds(0, NPOS)])
        pltpu.sync_copy(sf1_hbm.at[pl.ds(e0, NPOS)], sf1.at[pl.ds(0, NPOS)])
        pltpu.sync_copy(bd0_hbm.at[pl.ds(e0, NPOS)], bd0.at[pl.ds(0, NPOS)])
        pltpu.sync_copy(bd1_hbm.at[pl.ds(e0, NPOS)], bd1.at[pl.ds(0, NPOS)])

        for g in range(NGRP):
            f = 16 * g
            s0 = sf0[pl.ds(f, 16)]
            s1 = sf1[pl.ds(f, 16)]
            c0 = bd0[pl.ds(f, 16)]
            c1 = bd1[pl.ds(f, 16)]
            p0 = jnp.where(c0 > 0, PD, s0)
            p1 = jnp.where(c1 > 0, PD, s1)
            iv = p1 * (PD + 1) + p0
            idxb[f // 128, pl.ds(f % 128, 16)] = iv

        @pl.loop(0, NIDXROW)
        def _fire(i):
            pltpu.make_async_copy(
                tab_hbm.at[idxb.at[i]], rows.at[pl.ds(128 * i, 128)], sem
            ).start()

        @pl.loop(0, NIDXROW)
        def _drain(i):
            pltpu.make_async_copy(
                tab_hbm.at[idxb.at[i]], rows.at[pl.ds(128 * i, 128)], sem
            ).wait()

        for img in range(CHUNK):
            rbase = img * P
            rstore = img * 16 + iota

            @pl.loop(0, NT)
            def _tile(t):
                p0 = 16 * t
                for s in range(16):
                    perm = lax.bitwise_and(iota + s, 15)
                    rload = (rbase + p0) + perm
                    val = plsc.load_gather(rows, [rload, iota])
                    cvec = p0 + perm
                    plsc.store_scatter(outv, [rstore, cvec], val,
                                       mask=cvec < P)

        @pl.loop(0, CHUNK)
        def _out(img):
            pltpu.sync_copy(
                outv.at[pl.ds(16 * img, 16)],
                out_hbm.at[b0 + img],
            )


@jax.jit
def _pcode_embed(sparse_feature_input, board_input, pcode_embedding):
    sf0 = sparse_feature_input[:, 10].reshape(B * P)
    sf1 = sparse_feature_input[:, 11].reshape(B * P)
    bd0 = board_input[:, 0].reshape(B * P)
    bd1 = board_input[:, 1].reshape(B * P)
    mesh = plsc.VectorSubcoreMesh(core_axis_name="c", subcore_axis_name="s")
    run = pl.kernel(
        _sc_kernel,
        out_type=jax.ShapeDtypeStruct((B, D, P), jnp.float32),
        mesh=mesh,
        scratch_types=[
            pltpu.VMEM((NPAD,), jnp.int32),
            pltpu.VMEM((NPAD,), jnp.int32),
            pltpu.VMEM((NPAD,), jnp.int32),
            pltpu.VMEM((NPAD,), jnp.int32),
            pltpu.VMEM((NIDXROW, 128), jnp.int32),
            pltpu.VMEM((NROWS, D), jnp.float32),
            pltpu.VMEM((CHUNK * D, P), jnp.float32),
            pltpu.SemaphoreType.DMA,
        ],
        compiler_params=pltpu.CompilerParams(
            use_tc_tiling_on_sc=False, needs_layout_passes=False
        ),
    )
    out = run(sf0, sf1, bd0, bd1, pcode_embedding)
    return out.reshape(B, D, H, W)


def kernel(sparse_feature_dim, sparse_feature_input, board_input, pcode_embedding):
    del sparse_feature_dim
    return _pcode_embed(sparse_feature_input, board_input, pcode_embedding)


# speedup vs baseline: 3.2057x; 1.0024x over previous
import jax
import jax.numpy as jnp
from jax import lax
from jax.experimental import pallas as pl
from jax.experimental.pallas import tpu as pltpu
from jax.experimental.pallas import tpu_sc as plsc

B = 4096
H = 15
W = 15
P = H * W
D = 16
PD = 2380
VOCAB = (PD + 1) ** 2

NC = 2
NS = 16
NW = NC * NS
IMG_PER_W = B // NW
CHUNK = 8
NCHUNK = IMG_PER_W // CHUNK
NPOS = CHUNK * P
NGRP = 113
NPAD = 1808
NIDXROW = 15
NROWS = NIDXROW * 128
NT = 15


def _sc_kernel(sf0_hbm, sf1_hbm, bd0_hbm, bd1_hbm, tab_hbm, out_hbm,
               stg, idxb, rows, outv, semS, semO, semG0, semG1):
    wid = lax.axis_index("s") * NC + lax.axis_index("c")
    base_img = wid * IMG_PER_W

    iota = lax.broadcasted_iota(jnp.int32, (16,), 0)
    zeros16 = jnp.zeros((16,), jnp.int32)
    chans = [sf0_hbm, sf1_hbm, bd0_hbm, bd1_hbm]

    for q in range(2):
        for ch in range(4):
            stg[q, ch, pl.ds(1792, 16)] = zeros16
        for gg in range(1, 8):
            idxb[q, NIDXROW - 1, pl.ds(16 * gg, 16)] = zeros16

    def stage_copies(c, q):
        e0 = (base_img + c * CHUNK) * P
        return [
            pltpu.make_async_copy(
                chans[ch].at[pl.ds(e0, NPOS)],
                stg.at[q, ch, pl.ds(0, NPOS)],
                semS,
            )
            for ch in range(4)
        ]

    def fire_stage(c, q):
        for cp in stage_copies(c, q):
            cp.start()

    def drain_stage(c, q):
        for cp in stage_copies(c, q):
            cp.wait()

    def build_idx(q):
        @pl.loop(0, NGRP // 4)
        def _grp(g4):
            for u in range(4):
                f = 64 * g4 + 16 * u
                s0 = stg[q, 0, pl.ds(f, 16)]
                s1 = stg[q, 1, pl.ds(f, 16)]
                c0 = stg[q, 2, pl.ds(f, 16)]
                c1 = stg[q, 3, pl.ds(f, 16)]
                p0 = jnp.where(c0 > 0, PD, s0)
                p1 = jnp.where(c1 > 0, PD, s1)
                idxb[q, lax.shift_right_logical(f, 7),
                     pl.ds(lax.bitwise_and(f, 127), 16)] = p1 * (PD + 1) + p0
        f = 16 * (NGRP - 1)
        s0 = stg[q, 0, pl.ds(f, 16)]
        s1 = stg[q, 1, pl.ds(f, 16)]
        c0 = stg[q, 2, pl.ds(f, 16)]
        c1 = stg[q, 3, pl.ds(f, 16)]
        p0 = jnp.where(c0 > 0, PD, s0)
        p1 = jnp.where(c1 > 0, PD, s1)
        idxb[q, f // 128, pl.ds(f % 128, 16)] = p1 * (PD + 1) + p0

    def gather_copies(q, sem):
        return [
            pltpu.make_async_copy(
                tab_hbm.at[idxb.at[q, i]],
                rows.at[q, pl.ds(128 * i, 128)],
                sem,
            )
            for i in range(NIDXROW)
        ]

    def fire_gathers(q, sem):
        for cp in gather_copies(q, sem):
            cp.start()

    def drain_gathers(q, sem):
        for cp in gather_copies(q, sem):
            cp.wait()

    def transpose(q):
        for img in range(CHUNK):
            rbase = img * P
            rstore = img * 16 + iota

            @pl.loop(0, NT)
            def _tile(t):
                p0 = 16 * t
                for s in range(16):
                    perm = lax.bitwise_and(iota + s, 15)
                    rload = (rbase + p0) + perm
                    val = plsc.load_gather(rows.at[q], [rload, iota])
                    cvec = p0 + perm
                    plsc.store_scatter(outv, [rstore, cvec], val,
                                       mask=cvec < P)

    def out_copies(c):
        b0 = base_img + c * CHUNK
        return [
            pltpu.make_async_copy(
                outv.at[pl.ds(16 * img, 16)],
                out_hbm.at[b0 + img],
                semO,
            )
            for img in range(CHUNK)
        ]

    def fire_outs(c):
        for cp in out_copies(c):
            cp.start()

    def drain_outs(c):
        for cp in out_copies(c):
            cp.wait()

    def body(c, q, semG, semGo):
        @pl.when(c > 0)
        def _do():
            drain_outs(c - 1)

        @pl.when(c + 1 < NCHUNK)
        def _prep():
            drain_stage(c + 1, 1 - q)
            build_idx(1 - q)
            fire_gathers(1 - q, semGo)

        @pl.when(c + 2 < NCHUNK)
        def _stage2():
            fire_stage(c + 2, q)

        drain_gathers(q, semG)
        transpose(q)
        fire_outs(c)

    fire_stage(0, 0)
    drain_stage(0, 0)
    build_idx(0)
    fire_gathers(0, semG0)
    fire_stage(1, 1)

    @pl.loop(0, NCHUNK // 2)
    def _dp(dp):
        body(2 * dp, 0, semG0, semG1)
        body(2 * dp + 1, 1, semG1, semG0)

    drain_outs(NCHUNK - 1)


@jax.jit
def _pcode_embed(sparse_feature_input, board_input, pcode_embedding):
    sf0 = sparse_feature_input[:, 10].reshape(B * P)
    sf1 = sparse_feature_input[:, 11].reshape(B * P)
    bd0 = board_input[:, 0].reshape(B * P)
    bd1 = board_input[:, 1].reshape(B * P)
    mesh = plsc.VectorSubcoreMesh(core_axis_name="c", subcore_axis_name="s")
    run = pl.kernel(
        _sc_kernel,
        out_type=jax.ShapeDtypeStruct((B, D, P), jnp.float32),
        mesh=mesh,
        scratch_types=[
            pltpu.VMEM((2, 4, NPAD), jnp.int32),
            pltpu.VMEM((2, NIDXROW, 128), jnp.int32),
            pltpu.VMEM((2, NROWS, D), jnp.float32),
            pltpu.VMEM((CHUNK * D, P), jnp.float32),
            pltpu.SemaphoreType.DMA,
            pltpu.SemaphoreType.DMA,
            pltpu.SemaphoreType.DMA,
            pltpu.SemaphoreType.DMA,
        ],
        compiler_params=pltpu.CompilerParams(
            use_tc_tiling_on_sc=False, needs_layout_passes=False
        ),
    )
    out = run(sf0, sf1, bd0, bd1, pcode_embedding)
    return out.reshape(B, D, H, W)


def kernel(sparse_feature_dim, sparse_feature_input, board_input, pcode_embedding):
    del sparse_feature_dim
    return _pcode_embed(sparse_feature_input, board_input, pcode_embedding)
